# baseline (speedup 1.0000x reference)
import functools

import jax
import jax.numpy as jnp
import numpy as np
from jax import lax
from jax.experimental import pallas as pl
from jax.experimental.pallas import tpu as pltpu
from jax.experimental.pallas import tpu_sc as plsc

_NUM_CLASSES = 3
_B, _N, _M = 32, 256, 1024
_N_CHUNK = 128

_KS0 = 0
_KS1 = 12345
_KS2 = _KS0 ^ _KS1 ^ 0x1BD11BDA

_ROT0 = (13, 15, 26, 6)
_ROT1 = (17, 29, 16, 24)


def _rotl(x, d):
    return (x << jnp.uint32(d)) | (x >> jnp.uint32(32 - d))


def _four_rounds(x0, x1, rots):
    for r in rots:
        x0 = x0 + x1
        x1 = _rotl(x1, r)
        x1 = x1 ^ x0
    return x0, x1


def _threefry_xored(j):
    ks0 = jnp.uint32(_KS0)
    ks1 = jnp.uint32(_KS1)
    ks2 = jnp.uint32(_KS2)
    x1 = j + ks1
    x0 = x1
    x1 = _rotl(x1, _ROT0[0])
    x1 = x1 ^ x0
    for r in _ROT0[1:]:
        x0 = x0 + x1
        x1 = _rotl(x1, r)
        x1 = x1 ^ x0
    x0 = x0 + ks1
    x1 = x1 + (ks2 + jnp.uint32(1))
    x0, x1 = _four_rounds(x0, x1, _ROT1)
    x0 = x0 + ks2
    x1 = x1 + (ks0 + jnp.uint32(2))
    x0, x1 = _four_rounds(x0, x1, _ROT0)
    x0 = x0 + ks0
    x1 = x1 + (ks1 + jnp.uint32(3))
    x0, x1 = _four_rounds(x0, x1, _ROT1)
    x0 = x0 + ks1
    x1 = x1 + (ks2 + jnp.uint32(4))
    x0, x1 = _four_rounds(x0, x1, _ROT0)
    x0 = x0 + ks2
    x1 = x1 + (ks0 + jnp.uint32(5))
    return x0 ^ x1


def _neglog_u_from_bits(bits):
    tiny = jnp.float32(jnp.finfo(jnp.float32).tiny)
    one = jnp.float32(1.0)
    float_bits = (bits >> jnp.uint32(9)) | jnp.uint32(0x3F800000)
    floats = jax.lax.bitcast_convert_type(float_bits, jnp.float32) - one
    u = jnp.maximum(tiny, floats * (one - tiny) + tiny)
    return -jnp.log(u)


_CH = 16


def _body(t_ref, x0_ref, mask_ref, qbar_ref, out_ref):
    b = pl.program_id(0)
    nc = pl.program_id(1)

    qrow = qbar_ref[0].astype(jnp.bfloat16).astype(jnp.float32)
    winv = 1.0 / jnp.clip(qrow, 1e-20, None)
    wsl = [jax.lax.slice(winv, (0, i), (1, i + 1)) for i in range(9)]

    m = mask_ref[0]

    shape = (_CH, _M)
    ni = jax.lax.broadcasted_iota(jnp.uint32, shape, 0)
    mi = jax.lax.broadcasted_iota(jnp.uint32, shape, 1)
    n0 = (nc * _N_CHUNK).astype(jnp.uint32)
    row0 = b.astype(jnp.uint32) * jnp.uint32(_N) + n0

    def bcast(row, col):
        return jax.lax.broadcast_in_dim(wsl[3 * row + col], shape, (0, 1))

    j0_init = ((row0 + ni) * jnp.uint32(_M) + mi) * jnp.uint32(3)

    def chunk(i, j0):
        x = x0_ref[0, pl.ds(i * _CH, _CH), :]

        is1 = x == 1
        is2 = x == 2

        s = []
        for c in range(_NUM_CLASSES):
            e = _neglog_u_from_bits(_threefry_xored(j0 + jnp.uint32(c)))
            wc = jnp.where(is2, bcast(2, c), jnp.where(is1, bcast(1, c), bcast(0, c)))
            s.append(e * wc)

        idx = jnp.where(s[1] < s[0], 1, 0).astype(jnp.int32)
        sm = jnp.minimum(s[0], s[1])
        res = jnp.where(s[2] < sm, 2, idx).astype(jnp.int32)

        out_ref[0, pl.ds(i * _CH, _CH), :] = jnp.where(m != 0, res, 0)
        return j0 + jnp.uint32(3 * _CH * _M)

    jax.lax.fori_loop(0, _N_CHUNK // _CH, chunk, j0_init, unroll=8)


_B_SC = 8
_B_TC = _B - _B_SC
_PIECE = 8192
_LN2 = 0.6931471805599453

def _neglog_poly(u):
    ui = lax.bitcast_convert_type(u, jnp.int32)
    k = lax.shift_right_logical(ui, jnp.int32(23)) - jnp.int32(127)
    f = lax.bitcast_convert_type(
        (ui & jnp.int32(0x7FFFFF)) | jnp.int32(0x3F800000), jnp.float32)
    big = f > jnp.float32(1.4142135)
    f = jnp.where(big, f * jnp.float32(0.5), f)
    k = jnp.where(big, k + jnp.int32(1), k)
    z = (f - jnp.float32(1.0)) / (f + jnp.float32(1.0))
    z2 = z * z
    p = jnp.float32(1.0 / 9.0)
    p = p * z2 + jnp.float32(1.0 / 7.0)
    p = p * z2 + jnp.float32(1.0 / 5.0)
    p = p * z2 + jnp.float32(1.0 / 3.0)
    p = p * z2 + jnp.float32(1.0)
    logf = jnp.float32(2.0) * z * p
    return -(k.astype(jnp.float32) * jnp.float32(_LN2) + logf)


def _sc_body(x_ref, t_ref, mask_ref, q16_ref, lanes_ref, out_ref,
             idx_v, qrow_v, w_v, mask_v, x_v, o_v, lanes_v, sem):
    wid = lax.axis_index("s") * 2 + lax.axis_index("c")

    batch = wid // 4
    quarter = wid % 4
    cell0 = (batch * _N + quarter * (_N // 4)) * _M

    pltpu.sync_copy(t_ref.at[batch], idx_v)
    pltpu.async_copy(q16_ref.at[idx_v], qrow_v, sem).wait()
    pltpu.sync_copy(lanes_ref, lanes_v)
    qi = lax.bitcast_convert_type(qrow_v[0, pl.ds(0, 16)], jnp.int32)
    qi = (qi + jnp.int32(0x7FFF)
          + (lax.shift_right_logical(qi, jnp.int32(16)) & jnp.int32(1))) & jnp.int32(-65536)
    qrow = lax.bitcast_convert_type(qi, jnp.float32)
    w_v[...] = jnp.float32(1.0) / jnp.maximum(qrow, jnp.float32(1e-20))
    pltpu.sync_copy(mask_ref.at[batch], mask_v)

    tiny = jnp.float32(jnp.finfo(jnp.float32).tiny)
    n_pieces = (_N // 4) * _M // _PIECE

    def piece(p, _):
        base = cell0 + p * _PIECE
        pltpu.sync_copy(x_ref.at[pl.ds(base, _PIECE)], x_v)

        def vreg(i, _):
            lanes = lanes_v[...]
            x = x_v[pl.ds(i * 16, 16)]
            j0 = ((jnp.int32(_B_TC * _N * _M) + base + i * 16 + lanes)
                  * jnp.int32(3)).astype(jnp.uint32)
            s = []
            for c in range(_NUM_CLASSES):
                bits = _threefry_xored(j0 + jnp.uint32(c))
                fb = lax.shift_right_logical(bits, jnp.uint32(9)) | jnp.uint32(0x3F800000)
                u = lax.bitcast_convert_type(fb, jnp.float32) - jnp.float32(1.0)
                e = _neglog_poly(jnp.maximum(u, tiny))
                wc = plsc.load_gather(w_v, [x * 3 + c])
                s.append(e * wc)
            idx01 = jnp.where(s[1] < s[0], 1, 0).astype(jnp.int32)
            sm = jnp.minimum(s[0], s[1])
            res = jnp.where(s[2] < sm, 2, idx01).astype(jnp.int32)
            mk = plsc.load_gather(mask_v, [(i * 16) % _M + lanes])
            o_v[pl.ds(i * 16, 16)] = jnp.where(mk != 0, res, 0)
            return 0

        lax.fori_loop(0, _PIECE // 16, vreg, 0)
        pltpu.sync_copy(o_v, out_ref.at[pl.ds(base, _PIECE)])
        return 0

    lax.fori_loop(0, n_pieces, piece, 0)


def _sc_sample(x_sc_flat, t_sc, mask_sc, q16):
    mesh = plsc.VectorSubcoreMesh(core_axis_name="c", subcore_axis_name="s")
    return pl.kernel(
        _sc_body,
        out_type=jax.ShapeDtypeStruct((_B_SC * _N * _M,), jnp.int32),
        mesh=mesh,
        compiler_params=pltpu.CompilerParams(needs_layout_passes=False),
        scratch_types=[
            pltpu.VMEM((16,), jnp.int32),
            pltpu.VMEM((16, 128), jnp.float32),
            pltpu.VMEM((16,), jnp.float32),
            pltpu.VMEM((_M,), jnp.int32),
            pltpu.VMEM((_PIECE,), jnp.int32),
            pltpu.VMEM((_PIECE,), jnp.int32),
            pltpu.VMEM((16,), jnp.int32),
            pltpu.SemaphoreType.DMA,
        ],
    )(x_sc_flat, t_sc, mask_sc, q16, jnp.arange(16, dtype=jnp.int32))


@jax.jit
def kernel(x_0, t, clause_mask, Q_bar_mats):
    x_0 = x_0.astype(jnp.int32)
    t = t.astype(jnp.int32)
    mask32 = clause_mask.astype(jnp.int32)
    mask = mask32[:_B_TC].reshape(_B_TC, 1, _M)
    qbar = Q_bar_mats.astype(jnp.float32).reshape(1000, 1, 9)

    n_chunks = _N // _N_CHUNK
    grid = (_B_TC, n_chunks)

    out_tc = pl.pallas_call(
        _body,
        grid_spec=pltpu.PrefetchScalarGridSpec(
            num_scalar_prefetch=1,
            grid=grid,
            in_specs=[
                pl.BlockSpec((1, _N_CHUNK, _M), lambda b, nc, t_ref: (b, nc, 0)),
                pl.BlockSpec((1, 1, _M), lambda b, nc, t_ref: (b, 0, 0)),
                pl.BlockSpec((1, 1, 9), lambda b, nc, t_ref: (t_ref[b], 0, 0)),
            ],
            out_specs=pl.BlockSpec((1, _N_CHUNK, _M), lambda b, nc, t_ref: (b, nc, 0)),
        ),
        out_shape=jax.ShapeDtypeStruct((_B_TC, _N, _M), jnp.int32),
        compiler_params=pltpu.CompilerParams(
            dimension_semantics=("parallel", "parallel"),
        ),
    )(t, x_0[:_B_TC], mask, qbar)

    q16 = jnp.concatenate(
        [Q_bar_mats.astype(jnp.float32).reshape(1000, 9),
         jnp.ones((1000, 119), jnp.float32)], axis=1)
    t16 = jnp.broadcast_to(t[_B_TC:, None], (_B_SC, 16))
    out_sc = _sc_sample(
        x_0[_B_TC:].reshape(-1), t16, mask32[_B_TC:], q16)

    return jnp.concatenate([out_tc, out_sc.reshape(_B_SC, _N, _M)], axis=0)
